# baseline (speedup 1.0000x reference)
import jax
import jax.numpy as jnp
from jax.experimental import pallas as pl
from jax.experimental.pallas import tpu as pltpu

M = 65536
B = 256


def _dma_body(i0_ref, i1_ref, o0_ref, o1_ref, sem0, sem1):
    c0 = pltpu.make_async_copy(i0_ref, o0_ref, sem0)
    c1 = pltpu.make_async_copy(i1_ref, o1_ref, sem1)
    c0.start()
    c1.start()
    c0.wait()
    c1.wait()


def kernel(input0, input1, input2, input3):
    del input2, input3
    anyspec = pl.BlockSpec(memory_space=pl.ANY)
    out0, out1 = pl.pallas_call(
        _dma_body,
        in_specs=[anyspec, anyspec],
        out_specs=[anyspec, anyspec],
        out_shape=[
            jax.ShapeDtypeStruct((M, B), jnp.float32),
            jax.ShapeDtypeStruct((M, B), jnp.float32),
        ],
        scratch_shapes=[pltpu.SemaphoreType.DMA, pltpu.SemaphoreType.DMA],
    )(input0, input1)
    return (out0, out1)


# speedup vs baseline: 47.9466x; 47.9466x over previous
import jax
import jax.numpy as jnp
from jax.experimental import pallas as pl
from jax.experimental.pallas import tpu as pltpu

M = 65536
B = 256
BM = 2048


def _copy_body(i0_ref, i1_ref, o0_ref, o1_ref):
    o0_ref[...] = i0_ref[...]
    o1_ref[...] = i1_ref[...]


def kernel(input0, input1, input2, input3):
    del input2, input3
    grid = (M // BM,)
    spec = pl.BlockSpec((BM, B), lambda i: (i, 0))
    out0, out1 = pl.pallas_call(
        _copy_body,
        grid=grid,
        in_specs=[spec, spec],
        out_specs=[spec, spec],
        out_shape=[
            jax.ShapeDtypeStruct((M, B), jnp.float32),
            jax.ShapeDtypeStruct((M, B), jnp.float32),
        ],
        compiler_params=pltpu.CompilerParams(
            dimension_semantics=("parallel",),
        ),
    )(input0, input1)
    return (out0, out1)
